# Initial kernel scaffold; baseline (speedup 1.0000x reference)
import jax
import jax.numpy as jnp
from jax.experimental import pallas as pl


def kernel(x):
    raise NotImplementedError("write your pallas kernel here")


# speedup vs baseline: 2.4223x; 2.4223x over previous
import functools
import math

import jax
import jax.numpy as jnp
from jax import lax
from jax.experimental import pallas as pl
from jax.experimental.pallas import tpu as pltpu
from jax.experimental.pallas import tpu_sc as plsc

_B, _T, _C = 2, 8192, 768
_T1 = _T // 2
_R = math.floor(_T - _T * 0.95)
_U = _T1 - _R
_OUT = _U + _T1
_BT = 256
_H = _C // 2
_NT = 16
_TPT = _T1 // (2 * _NT) * 2
_CH = 128


def _score_kernel(a_ref, b_ref, nmax_ref, nidx_ref):
    a = a_ref[0, :, :_C]
    b = b_ref[0, :, _C:]
    scores = jax.lax.dot_general(
        a, b, (((1,), (1,)), ((), ())),
        preferred_element_type=jnp.float32,
    )
    nmax_ref[0, 0, 0, :] = jnp.max(scores, axis=-1)
    nidx_ref[0, 0, 0, :] = jnp.argmax(scores, axis=-1).astype(jnp.int32)


def _scores(xn2):
    grid = (_B, _T1 // _BT)
    return pl.pallas_call(
        _score_kernel,
        grid=grid,
        in_specs=[
            pl.BlockSpec((1, _BT, 2 * _C), lambda b, i: (b, i, 0)),
            pl.BlockSpec((1, _T1, 2 * _C), lambda b, i: (b, 0, 0)),
        ],
        out_specs=[
            pl.BlockSpec((1, 1, 1, _BT), lambda b, i: (b, i, 0, 0)),
            pl.BlockSpec((1, 1, 1, _BT), lambda b, i: (b, i, 0, 0)),
        ],
        out_shape=[
            jax.ShapeDtypeStruct((_B, _T1 // _BT, 1, _BT), jnp.float32),
            jax.ShapeDtypeStruct((_B, _T1 // _BT, 1, _BT), jnp.int32),
        ],
    )(xn2, xn2)


def _meta_kernel(nmc_ref, nmr_ref, nic_ref, dest2_ref, route_ref):
    col = nmc_ref[0]
    row = nmr_ref[0]
    ni = nic_ref[0]
    b = pl.program_id(0)
    i0 = pl.program_id(1) * _BT
    gt = (row > col).astype(jnp.int32)
    jidx = lax.broadcasted_iota(jnp.int32, (_BT, _T1), 1)
    iabs = i0 + lax.broadcasted_iota(jnp.int32, (_BT, _T1), 0)
    tie = ((row == col) & (jidx < iabs)).astype(jnp.int32)
    rank = jnp.sum(gt + tie, axis=1, keepdims=True)
    sel = rank < _R
    dest = jnp.where(sel, _U + ni, rank - _R) + b * _OUT
    dest2_ref[0, 0] = 2 * dest
    route_ref[0, 0] = jnp.where(sel, ni, _T1)


def _meta(node_max, node_idx):
    nmc = node_max.reshape(_B, _T1, 1)
    nmr = node_max.reshape(_B, 1, _T1)
    nic = node_idx.reshape(_B, _T1, 1)
    grid = (_B, _T1 // _BT)
    dest2, route = pl.pallas_call(
        _meta_kernel,
        grid=grid,
        in_specs=[
            pl.BlockSpec((1, _BT, 1), lambda b, i: (b, i, 0)),
            pl.BlockSpec((1, 1, _T1), lambda b, i: (b, 0, 0)),
            pl.BlockSpec((1, _BT, 1), lambda b, i: (b, i, 0)),
        ],
        out_specs=[
            pl.BlockSpec((1, 1, _BT, 1), lambda b, i: (b, i, 0, 0)),
            pl.BlockSpec((1, 1, _BT, 1), lambda b, i: (b, i, 0, 0)),
        ],
        out_shape=[
            jax.ShapeDtypeStruct((_B, _T1 // _BT, _BT, 1), jnp.int32),
            jax.ShapeDtypeStruct((_B, _T1 // _BT, _BT, 1), jnp.int32),
        ],
    )(nmc, nmr, nic)
    return dest2.reshape(_B * _T1), route.reshape(_B * _T1)


def _iota16():
    return lax.broadcasted_iota(jnp.int32, (16,), 0)


def _fill_idx(idx_ref, base, stride):
    for g in range(_CH // 16):
        idx_ref[pl.ds(g * 16, 16)] = base + stride * (g * 16 + _iota16())


def _sc_merge_body(xh, dest2h, routeh, out, buf, acc, idxv, destv, routev,
                   mlist, cnt, sem):
    c = lax.axis_index("c")
    s = lax.axis_index("s")
    ones16 = jnp.full((16,), 1.0, jnp.float32)

    for b in range(_B):
        for k2 in range(_TPT // _CH):
            t0 = s * _TPT + k2 * _CH
            _fill_idx(idxv, 4 * (b * _T1 + t0) + c, 4)
            pltpu.async_copy(xh.at[idxv], buf, sem).wait()
            pltpu.sync_copy(dest2h.at[pl.ds(b * _T1 + t0, _CH)], destv)
            for g in range(_CH // 16):
                destv[pl.ds(g * 16, 16)] = destv[pl.ds(g * 16, 16)] + c
            pltpu.sync_copy(buf, out.at[destv])
        plsc.subcore_barrier()

        pltpu.sync_copy(routeh.at[pl.ds(b * _T1, _T1)], routev)
        for k2 in range(_TPT // _CH):
            j0 = s * _TPT + k2 * _CH
            _fill_idx(idxv, 4 * (b * _T1 + j0) + 2 + c, 4)
            pltpu.async_copy(xh.at[idxv], acc, sem).wait()
            for g in range(_CH // 16):
                cnt[pl.ds(g * 16, 16)] = jnp.zeros((16,), jnp.float32)

            def scan_step(t, off):
                r16 = routev[pl.ds(t * 16, 16)]
                local = r16 - j0
                m = (local >= 0) & (local < _CH)
                plsc.addupdate_scatter(cnt, [local], ones16, mask=m)
                packed = local * 4096 + (t * 16 + _iota16())
                plsc.store_compressed(mlist.at[pl.ds(off, 16)], packed, mask=m)
                npc = plsc.all_reduce_population_count(m)
                return off + jnp.max(npc, axis=0)

            nm = lax.fori_loop(0, _T1 // 16, scan_step, 0)

            def chunk_body(q, _):
                base = q * _CH
                for g in range(_CH // 16):
                    pk = mlist[pl.ds(base + g * 16, 16)]
                    tok = jnp.bitwise_and(pk, 4095)
                    idxv[pl.ds(g * 16, 16)] = 4 * (b * _T1 + tok) + c
                pltpu.async_copy(xh.at[idxv], buf, sem).wait()
                here = jnp.minimum(_CH, nm - base)

                def add_row(k, _):
                    k16 = jnp.zeros((16,), jnp.int32) + k
                    pk16 = plsc.load_gather(mlist, [base + k16])
                    lm16 = jnp.right_shift(pk16, 12)
                    for j in range(_H // 16):
                        col = j * 16 + _iota16()
                        vals = plsc.load_gather(buf, [k16, col])
                        plsc.addupdate_scatter(acc, [lm16, col], vals)
                    return 0

                lax.fori_loop(0, here, add_row, 0)
                return 0

            lax.fori_loop(0, (nm + _CH - 1) // _CH, chunk_body, 0)

            def scale_row(k, _):
                k16 = jnp.zeros((16,), jnp.int32) + k
                c16 = plsc.load_gather(cnt, [k16])
                inv16 = 1.0 / (1.0 + c16)
                for j in range(_H // 16):
                    col = j * 16 + _iota16()
                    v = plsc.load_gather(acc, [k16, col]) * inv16
                    plsc.store_scatter(acc, [k16, col], v)
                return 0

            lax.fori_loop(0, _CH, scale_row, 0)
            _fill_idx(idxv, 2 * (b * _OUT + _U + j0) + c, 2)
            pltpu.sync_copy(acc, out.at[idxv])


def _sc_merge(xh, dest2, route):
    mesh = plsc.VectorSubcoreMesh(core_axis_name="c", subcore_axis_name="s")
    kfn = functools.partial(
        pl.kernel,
        mesh=mesh,
        out_type=jax.ShapeDtypeStruct((_B * _OUT * 2, _H), jnp.float32),
        compiler_params=pltpu.CompilerParams(needs_layout_passes=False),
        scratch_types=[
            pltpu.VMEM((_CH, _H), jnp.float32),
            pltpu.VMEM((_CH, _H), jnp.float32),
            pltpu.VMEM((_CH,), jnp.int32),
            pltpu.VMEM((_CH,), jnp.int32),
            pltpu.VMEM((_T1,), jnp.int32),
            pltpu.VMEM((544,), jnp.int32),
            pltpu.VMEM((_CH,), jnp.float32),
            pltpu.SemaphoreType.DMA,
        ],
    )(_sc_merge_body)
    return kfn(xh, dest2, route)


def kernel(x):
    n = jnp.linalg.norm(x, axis=-1, keepdims=True)
    xn = x / jnp.maximum(n, 1e-12)
    xn2 = xn.reshape(_B, _T1, 2 * _C)
    node_max, node_idx = _scores(xn2)
    node_max = node_max.reshape(_B, _T1)
    node_idx = node_idx.reshape(_B, _T1)
    dest2, route = _meta(node_max, node_idx)
    xh = x.reshape(_B * _T1 * 2 * 2, _H)
    out = _sc_merge(xh, dest2, route)
    return out.reshape(_B, _OUT, _C)


# speedup vs baseline: 2.6715x; 1.1029x over previous
import functools
import math

import jax
import jax.numpy as jnp
from jax import lax
from jax.experimental import pallas as pl
from jax.experimental.pallas import tpu as pltpu
from jax.experimental.pallas import tpu_sc as plsc

_B, _T, _C = 2, 8192, 768
_T1 = _T // 2
_R = math.floor(_T - _T * 0.95)
_U = _T1 - _R
_OUT = _U + _T1
_BT = 256
_NT = 16
_TPT = _T1 // _NT
_CH = 64


def _score_kernel(a_ref, b_ref, nmax_ref, nidx_ref):
    a = a_ref[0, :, :_C]
    b = b_ref[0, :, _C:]
    scores = jax.lax.dot_general(
        a, b, (((1,), (1,)), ((), ())),
        preferred_element_type=jnp.float32,
    )
    nmax_ref[0, 0, 0, :] = jnp.max(scores, axis=-1)
    nidx_ref[0, 0, 0, :] = jnp.argmax(scores, axis=-1).astype(jnp.int32)


def _scores(xn2):
    grid = (_B, _T1 // _BT)
    return pl.pallas_call(
        _score_kernel,
        grid=grid,
        in_specs=[
            pl.BlockSpec((1, _BT, 2 * _C), lambda b, i: (b, i, 0)),
            pl.BlockSpec((1, _T1, 2 * _C), lambda b, i: (b, 0, 0)),
        ],
        out_specs=[
            pl.BlockSpec((1, 1, 1, _BT), lambda b, i: (b, i, 0, 0)),
            pl.BlockSpec((1, 1, 1, _BT), lambda b, i: (b, i, 0, 0)),
        ],
        out_shape=[
            jax.ShapeDtypeStruct((_B, _T1 // _BT, 1, _BT), jnp.float32),
            jax.ShapeDtypeStruct((_B, _T1 // _BT, 1, _BT), jnp.int32),
        ],
    )(xn2, xn2)


def _meta_kernel(nmc_ref, nmr_ref, nic_ref, dest2_ref, route_ref):
    col = nmc_ref[0]
    row = nmr_ref[0]
    ni = nic_ref[0]
    b = pl.program_id(0)
    i0 = pl.program_id(1) * _BT
    gt = (row > col).astype(jnp.int32)
    jidx = lax.broadcasted_iota(jnp.int32, (_BT, _T1), 1)
    iabs = i0 + lax.broadcasted_iota(jnp.int32, (_BT, _T1), 0)
    tie = ((row == col) & (jidx < iabs)).astype(jnp.int32)
    rank = jnp.sum(gt + tie, axis=1, keepdims=True)
    sel = rank < _R
    dest2_ref[0, 0] = jnp.where(sel, _U + ni, rank - _R) + b * _OUT
    route_ref[0, 0] = jnp.where(sel, ni, _T1)


def _meta(node_max, node_idx):
    nmc = node_max.reshape(_B, _T1, 1)
    nmr = node_max.reshape(_B, 1, _T1)
    nic = node_idx.reshape(_B, _T1, 1)
    grid = (_B, _T1 // _BT)
    dest2, route = pl.pallas_call(
        _meta_kernel,
        grid=grid,
        in_specs=[
            pl.BlockSpec((1, _BT, 1), lambda b, i: (b, i, 0)),
            pl.BlockSpec((1, 1, _T1), lambda b, i: (b, 0, 0)),
            pl.BlockSpec((1, _BT, 1), lambda b, i: (b, i, 0)),
        ],
        out_specs=[
            pl.BlockSpec((1, 1, _BT, 1), lambda b, i: (b, i, 0, 0)),
            pl.BlockSpec((1, 1, _BT, 1), lambda b, i: (b, i, 0, 0)),
        ],
        out_shape=[
            jax.ShapeDtypeStruct((_B, _T1 // _BT, _BT, 1), jnp.int32),
            jax.ShapeDtypeStruct((_B, _T1 // _BT, _BT, 1), jnp.int32),
        ],
    )(nmc, nmr, nic)
    return dest2.reshape(_B * _T1), route.reshape(_B * _T1)


def _iota16():
    return lax.broadcasted_iota(jnp.int32, (16,), 0)


def _fill_idx(idx_ref, base, stride):
    for g in range(_CH // 16):
        idx_ref[pl.ds(g * 16, 16)] = base + stride * (g * 16 + _iota16())


def _sc_merge_body(xh, dest2h, routeh, out, buf, acc, idxv, destv, routev,
                   mlist, cnt, sem):
    b = lax.axis_index("c")
    s = lax.axis_index("s")
    ones16 = jnp.full((16,), 1.0, jnp.float32)

    for k2 in range(_TPT // _CH):
        t0 = s * _TPT + k2 * _CH
        _fill_idx(idxv, b * _T + 2 * t0, 2)
        pltpu.async_copy(xh.at[idxv], buf, sem).wait()
        pltpu.sync_copy(dest2h.at[pl.ds(b * _T1 + t0, _CH)], destv)
        pltpu.sync_copy(buf, out.at[destv])
    plsc.subcore_barrier()

    pltpu.sync_copy(routeh.at[pl.ds(b * _T1, _T1)], routev)
    for k2 in range(_TPT // _CH):
        j0 = s * _TPT + k2 * _CH
        _fill_idx(idxv, b * _T + 2 * j0 + 1, 2)
        pltpu.async_copy(xh.at[idxv], acc, sem).wait()
        for g in range(_CH // 16):
            cnt[pl.ds(g * 16, 16)] = jnp.zeros((16,), jnp.float32)

        def scan_step(t, off):
            r16 = routev[pl.ds(t * 16, 16)]
            local = r16 - j0
            m = (local >= 0) & (local < _CH)
            plsc.addupdate_scatter(cnt, [local], ones16, mask=m)
            packed = local * 4096 + (t * 16 + _iota16())
            plsc.store_compressed(mlist.at[pl.ds(off, 16)], packed, mask=m)
            npc = plsc.all_reduce_population_count(m)
            return off + jnp.max(npc, axis=0)

        nm = lax.fori_loop(0, _T1 // 16, scan_step, 0)

        def chunk_body(q, _):
            base = q * _CH
            for g in range(_CH // 16):
                pk = mlist[pl.ds(base + g * 16, 16)]
                tok = jnp.bitwise_and(pk, 4095)
                idxv[pl.ds(g * 16, 16)] = b * _T + 2 * tok
            pltpu.async_copy(xh.at[idxv], buf, sem).wait()
            here = jnp.minimum(_CH, nm - base)

            def add_row(k, _):
                k16 = jnp.zeros((16,), jnp.int32) + k
                pk16 = plsc.load_gather(mlist, [base + k16])
                lm16 = jnp.right_shift(pk16, 12)
                for j in range(_C // 16):
                    col = j * 16 + _iota16()
                    vals = plsc.load_gather(buf, [k16, col])
                    plsc.addupdate_scatter(acc, [lm16, col], vals)
                return 0

            lax.fori_loop(0, here, add_row, 0)
            return 0

        lax.fori_loop(0, (nm + _CH - 1) // _CH, chunk_body, 0)

        def scale_row(k, _):
            k16 = jnp.zeros((16,), jnp.int32) + k
            c16 = plsc.load_gather(cnt, [k16])
            inv16 = 1.0 / (1.0 + c16)
            for j in range(_C // 16):
                col = j * 16 + _iota16()
                v = plsc.load_gather(acc, [k16, col]) * inv16
                plsc.store_scatter(acc, [k16, col], v)
            return 0

        lax.fori_loop(0, _CH, scale_row, 0)
        _fill_idx(idxv, b * _OUT + _U + j0, 1)
        pltpu.sync_copy(acc, out.at[idxv])


def _sc_merge(xh, dest2, route):
    mesh = plsc.VectorSubcoreMesh(core_axis_name="c", subcore_axis_name="s")
    kfn = functools.partial(
        pl.kernel,
        mesh=mesh,
        out_type=jax.ShapeDtypeStruct((_B * _OUT, _C), jnp.float32),
        compiler_params=pltpu.CompilerParams(needs_layout_passes=False),
        scratch_types=[
            pltpu.VMEM((_CH, _C), jnp.float32),
            pltpu.VMEM((_CH, _C), jnp.float32),
            pltpu.VMEM((_CH,), jnp.int32),
            pltpu.VMEM((_CH,), jnp.int32),
            pltpu.VMEM((_T1,), jnp.int32),
            pltpu.VMEM((544,), jnp.int32),
            pltpu.VMEM((_CH,), jnp.float32),
            pltpu.SemaphoreType.DMA,
        ],
    )(_sc_merge_body)
    return kfn(xh, dest2, route)


def kernel(x):
    n = jnp.linalg.norm(x, axis=-1, keepdims=True)
    xn = x / jnp.maximum(n, 1e-12)
    xn2 = xn.reshape(_B, _T1, 2 * _C)
    node_max, node_idx = _scores(xn2)
    node_max = node_max.reshape(_B, _T1)
    node_idx = node_idx.reshape(_B, _T1)
    dest2, route = _meta(node_max, node_idx)
    xh = x.reshape(_B * _T, _C)
    out = _sc_merge(xh, dest2, route)
    return out.reshape(_B, _OUT, _C)


# speedup vs baseline: 2.7348x; 1.0237x over previous
import functools
import math

import jax
import jax.numpy as jnp
from jax import lax
from jax.experimental import pallas as pl
from jax.experimental.pallas import tpu as pltpu
from jax.experimental.pallas import tpu_sc as plsc

_B, _T, _C = 2, 8192, 768
_T1 = _T // 2
_R = math.floor(_T - _T * 0.95)
_U = _T1 - _R
_OUT = _U + _T1
_BT = 256
_NT = 16
_TPT = _T1 // _NT
_CH = 64


def _score_kernel(a_ref, b_ref, nmax_ref, nidx_ref):
    a = a_ref[0, :, :_C]
    b = b_ref[0, :, _C:]
    scoresT = jax.lax.dot_general(
        b, a, (((1,), (1,)), ((), ())),
        preferred_element_type=jnp.float32,
    )
    m = jnp.max(scoresT, axis=0)
    sidx = lax.broadcasted_iota(jnp.int32, scoresT.shape, 0)
    idx = jnp.min(jnp.where(scoresT == m[None, :], sidx, _T1), axis=0)
    nmax_ref[0, 0, 0, :] = m
    nidx_ref[0, 0, 0, :] = idx


def _scores(xn2):
    grid = (_B, _T1 // _BT)
    return pl.pallas_call(
        _score_kernel,
        grid=grid,
        in_specs=[
            pl.BlockSpec((1, _BT, 2 * _C), lambda b, i: (b, i, 0)),
            pl.BlockSpec((1, _T1, 2 * _C), lambda b, i: (b, 0, 0)),
        ],
        out_specs=[
            pl.BlockSpec((1, 1, 1, _BT), lambda b, i: (b, i, 0, 0)),
            pl.BlockSpec((1, 1, 1, _BT), lambda b, i: (b, i, 0, 0)),
        ],
        out_shape=[
            jax.ShapeDtypeStruct((_B, _T1 // _BT, 1, _BT), jnp.float32),
            jax.ShapeDtypeStruct((_B, _T1 // _BT, 1, _BT), jnp.int32),
        ],
    )(xn2, xn2)


def _meta_kernel(nmc_ref, nmr_ref, nic_ref, dest2_ref, route_ref):
    col = nmc_ref[0]
    row = nmr_ref[0]
    ni = nic_ref[0]
    b = pl.program_id(0)
    i0 = pl.program_id(1) * _BT
    gt = (row > col).astype(jnp.int32)
    jidx = lax.broadcasted_iota(jnp.int32, (_BT, _T1), 1)
    iabs = i0 + lax.broadcasted_iota(jnp.int32, (_BT, _T1), 0)
    tie = ((row == col) & (jidx < iabs)).astype(jnp.int32)
    rank = jnp.sum(gt + tie, axis=1, keepdims=True)
    sel = rank < _R
    dest2_ref[0, 0] = jnp.where(sel, _U + ni, rank - _R) + b * _OUT
    route_ref[0, 0] = jnp.where(sel, ni, _T1)


def _meta(node_max, node_idx):
    nmc = node_max.reshape(_B, _T1, 1)
    nmr = node_max.reshape(_B, 1, _T1)
    nic = node_idx.reshape(_B, _T1, 1)
    grid = (_B, _T1 // _BT)
    dest2, route = pl.pallas_call(
        _meta_kernel,
        grid=grid,
        in_specs=[
            pl.BlockSpec((1, _BT, 1), lambda b, i: (b, i, 0)),
            pl.BlockSpec((1, 1, _T1), lambda b, i: (b, 0, 0)),
            pl.BlockSpec((1, _BT, 1), lambda b, i: (b, i, 0)),
        ],
        out_specs=[
            pl.BlockSpec((1, 1, _BT, 1), lambda b, i: (b, i, 0, 0)),
            pl.BlockSpec((1, 1, _BT, 1), lambda b, i: (b, i, 0, 0)),
        ],
        out_shape=[
            jax.ShapeDtypeStruct((_B, _T1 // _BT, _BT, 1), jnp.int32),
            jax.ShapeDtypeStruct((_B, _T1 // _BT, _BT, 1), jnp.int32),
        ],
    )(nmc, nmr, nic)
    return dest2.reshape(_B * _T1), route.reshape(_B * _T1)


def _iota16():
    return lax.broadcasted_iota(jnp.int32, (16,), 0)


def _fill_idx(idx_ref, base, stride):
    for g in range(_CH // 16):
        idx_ref[pl.ds(g * 16, 16)] = base + stride * (g * 16 + _iota16())


def _sc_merge_body(xh, dest2h, routeh, out, buf, acc, idxv, destv, routev,
                   mlist, cnt, sem):
    b = lax.axis_index("c")
    s = lax.axis_index("s")
    ones16 = jnp.full((16,), 1.0, jnp.float32)

    for k2 in range(_TPT // _CH):
        t0 = s * _TPT + k2 * _CH
        _fill_idx(idxv, b * _T + 2 * t0, 2)
        pltpu.async_copy(xh.at[idxv], buf, sem).wait()
        pltpu.sync_copy(dest2h.at[pl.ds(b * _T1 + t0, _CH)], destv)
        pltpu.sync_copy(buf, out.at[destv])
    plsc.subcore_barrier()

    pltpu.sync_copy(routeh.at[pl.ds(b * _T1, _T1)], routev)
    for k2 in range(_TPT // _CH):
        j0 = s * _TPT + k2 * _CH
        _fill_idx(idxv, b * _T + 2 * j0 + 1, 2)
        pltpu.async_copy(xh.at[idxv], acc, sem).wait()
        for g in range(_CH // 16):
            cnt[pl.ds(g * 16, 16)] = jnp.zeros((16,), jnp.float32)

        def scan_step(t, off):
            r16 = routev[pl.ds(t * 16, 16)]
            local = r16 - j0
            m = (local >= 0) & (local < _CH)
            plsc.addupdate_scatter(cnt, [local], ones16, mask=m)
            packed = local * 4096 + (t * 16 + _iota16())
            plsc.store_compressed(mlist.at[pl.ds(off, 16)], packed, mask=m)
            npc = plsc.all_reduce_population_count(m)
            return off + jnp.max(npc, axis=0)

        nm = lax.fori_loop(0, _T1 // 16, scan_step, 0)

        def chunk_body(q, _):
            base = q * _CH
            for g in range(_CH // 16):
                pk = mlist[pl.ds(base + g * 16, 16)]
                tok = jnp.bitwise_and(pk, 4095)
                idxv[pl.ds(g * 16, 16)] = b * _T + 2 * tok
            pltpu.async_copy(xh.at[idxv], buf, sem).wait()
            here = jnp.minimum(_CH, nm - base)

            def add_row(k, _):
                k16 = jnp.zeros((16,), jnp.int32) + k
                pk16 = plsc.load_gather(mlist, [base + k16])
                lm16 = jnp.right_shift(pk16, 12)
                for j in range(_C // 16):
                    col = j * 16 + _iota16()
                    vals = plsc.load_gather(buf, [k16, col])
                    plsc.addupdate_scatter(acc, [lm16, col], vals)
                return 0

            lax.fori_loop(0, here, add_row, 0)
            return 0

        lax.fori_loop(0, (nm + _CH - 1) // _CH, chunk_body, 0)

        def scale_row(k, _):
            k16 = jnp.zeros((16,), jnp.int32) + k
            c16 = plsc.load_gather(cnt, [k16])
            inv16 = 1.0 / (1.0 + c16)
            for j in range(_C // 16):
                col = j * 16 + _iota16()
                v = plsc.load_gather(acc, [k16, col]) * inv16
                plsc.store_scatter(acc, [k16, col], v)
            return 0

        lax.fori_loop(0, _CH, scale_row, 0)
        _fill_idx(idxv, b * _OUT + _U + j0, 1)
        pltpu.sync_copy(acc, out.at[idxv])


def _sc_merge(xh, dest2, route):
    mesh = plsc.VectorSubcoreMesh(core_axis_name="c", subcore_axis_name="s")
    kfn = functools.partial(
        pl.kernel,
        mesh=mesh,
        out_type=jax.ShapeDtypeStruct((_B * _OUT, _C), jnp.float32),
        compiler_params=pltpu.CompilerParams(needs_layout_passes=False, use_tc_tiling_on_sc=True),
        scratch_types=[
            pltpu.VMEM((_CH, _C), jnp.float32),
            pltpu.VMEM((_CH, _C), jnp.float32),
            pltpu.VMEM((_CH,), jnp.int32),
            pltpu.VMEM((_CH,), jnp.int32),
            pltpu.VMEM((_T1,), jnp.int32),
            pltpu.VMEM((544,), jnp.int32),
            pltpu.VMEM((_CH,), jnp.float32),
            pltpu.SemaphoreType.DMA,
        ],
    )(_sc_merge_body)
    return kfn(xh, dest2, route)


def kernel(x):
    n = jnp.linalg.norm(x, axis=-1, keepdims=True)
    xn = (x / jnp.maximum(n, 1e-12)).astype(jnp.bfloat16)
    xn2 = xn.reshape(_B, _T1, 2 * _C)
    node_max, node_idx = _scores(xn2)
    node_max = node_max.reshape(_B, _T1)
    node_idx = node_idx.reshape(_B, _T1)
    dest2, route = _meta(node_max, node_idx)
    xh = x.reshape(_B * _T, _C)
    out = _sc_merge(xh, dest2, route)
    return out.reshape(_B, _OUT, _C)
